# baseline (speedup 1.0000x reference)
import functools

import jax
import jax.numpy as jnp
from jax import lax
from jax.experimental import pallas as pl
from jax.experimental.pallas import tpu as pltpu
from jax.experimental.pallas import tpu_sc as plsc

_B = 16384
_CTX = 20
_D = 64
_DI = _D // 2
_TOT = 2 * _B
_NC, _NS = 2, 16
_NW = _NC * _NS
_PER_W = _TOT // _NW
_G = 128
_E = 32
_GPC = _E * _CTX // _G
_CHUNKS = _PER_W // _E
_LANES = 16
_ROWS = 199999


_HALF = (_ROWS + 1) // 2


def _bf16_bits(t):
    b = lax.bitcast_convert_type(t, jnp.uint32)
    return (b + jnp.uint32(0x7FFF) + ((b >> 16) & jnp.uint32(1))) >> 16


def _pack_tables(u_table, w_table):
    pu = _bf16_bits(jnp.pad(u_table, ((0, 1), (0, 0))))
    pw = _bf16_bits(jnp.pad(w_table, ((0, 1), (0, 0))))
    iu = pu[:, 0::2] | (pu[:, 1::2] << 16)
    iw = pw[:, 0::2] | (pw[:, 1::2] << 16)
    comb = jnp.concatenate(
        [iu[:_HALF], iw[:_HALF], iu[_HALF:], iw[_HALF:]], axis=1)
    return lax.bitcast_convert_type(comb.reshape(2 * (_ROWS + 1), _DI),
                                    jnp.int32)


def _prep_indices(pos_u, pos_w, neg_u, neg_w):
    ui = jnp.concatenate(
        [pos_u.reshape(-1), neg_u.reshape(-1)]).astype(jnp.int32)
    wi = jnp.concatenate([pos_w, neg_w]).astype(jnp.int32)
    u_idx = (4 * (ui % _HALF) + 2 * (ui // _HALF)).reshape(
        _NW, _CHUNKS * _GPC, _G)
    w_idx = (4 * (wi % _HALF) + 2 * (wi // _HALF) + 1).reshape(
        _NW, _CHUNKS, _E)
    return u_idx, w_idx


def _sc_gather_sum(u_idx, w_idx, tab4):
    mesh = plsc.VectorSubcoreMesh(core_axis_name="c", subcore_axis_name="s")

    @functools.partial(
        pl.kernel,
        compiler_params=pltpu.CompilerParams(
            use_tc_tiling_on_sc=False, needs_layout_passes=False),
        out_type=jax.ShapeDtypeStruct((_TOT // 2, 4 * _DI), jnp.int32),
        mesh=mesh,
        scratch_types=[
            pltpu.VMEM((_CHUNKS * _GPC, _G), jnp.int32),
            pltpu.VMEM((_CHUNKS, _E), jnp.int32),
            pltpu.VMEM((_E * _CTX, _DI), jnp.int32),
            pltpu.VMEM((_E * _CTX, _DI), jnp.int32),
            pltpu.VMEM((_E, _DI), jnp.int32),
            pltpu.VMEM((_E, _DI), jnp.int32),
            pltpu.VMEM((_E // 2, 4 * _DI), jnp.int32),
            pltpu.VMEM((_E // 2, 4 * _DI), jnp.int32),
            pltpu.SemaphoreType.DMA,
            pltpu.SemaphoreType.DMA,
            pltpu.SemaphoreType.DMA,
            pltpu.SemaphoreType.DMA,
        ],
    )
    def k(uidx_hbm, widx_hbm, tab_hbm, out_hbm,
          uidx_v, widx_v, rows0, rows1, wrows0, wrows1, out0, out1,
          semg0, semg1, semo0, semo1):
        wid = lax.axis_index("s") * _NC + lax.axis_index("c")
        base2 = wid * _PER_W // 2
        pltpu.sync_copy(uidx_hbm.at[wid], uidx_v)
        pltpu.sync_copy(widx_hbm.at[wid], widx_v)

        def issue(ck, rows_v, wrows_v, semg):
            for j in range(_GPC):
                pltpu.async_copy(
                    tab_hbm.at[uidx_v.at[ck * _GPC + j]],
                    rows_v.at[pl.ds(j * _G, _G)],
                    semg,
                )
            pltpu.async_copy(tab_hbm.at[widx_v.at[ck]], wrows_v, semg)

        def drain(rows_v, wrows_v, semg):
            pltpu.make_async_copy(
                tab_hbm.at[pl.ds(0, _E * _CTX)], rows_v, semg).wait()
            pltpu.make_async_copy(tab_hbm.at[pl.ds(0, _E)], wrows_v, semg).wait()

        def compute(rows_v, wrows_v, out_v):
            @pl.loop(0, _E // 2)
            def _pair(p):
                for par in range(2):
                    r0 = (2 * p + par) * _CTX
                    ob = par * 2 * _DI
                    for h in range(_DI // _LANES):
                        sl = pl.ds(h * _LANES, _LANES)
                        acc = plsc.bitcast(rows_v[r0, sl], jnp.bfloat16)
                        for c in range(1, _CTX):
                            acc = acc + plsc.bitcast(
                                rows_v[r0 + c, sl], jnp.bfloat16)
                        out_v[p, pl.ds(ob + h * _LANES, _LANES)] = (
                            plsc.bitcast(acc, jnp.int32))
                        out_v[p, pl.ds(ob + _DI + h * _LANES, _LANES)] = (
                            wrows_v[2 * p + par, sl])

        def out_wait(out_v, semo):
            pltpu.make_async_copy(
                out_v, out_hbm.at[pl.ds(0, _E // 2)], semo).wait()

        _H = _CHUNKS // 2
        issue(0, rows0, wrows0, semg0)

        @pl.loop(0, _H)
        def _pipe(kk):
            ck0 = 2 * kk
            issue(ck0 + 1, rows1, wrows1, semg1)
            drain(rows0, wrows0, semg0)

            @pl.when(kk > 0)
            def _():
                out_wait(out0, semo0)

            compute(rows0, wrows0, out0)
            pltpu.async_copy(
                out0, out_hbm.at[pl.ds(base2 + ck0 * _E // 2, _E // 2)], semo0)

            @pl.when(kk < _H - 1)
            def _():
                issue(ck0 + 2, rows0, wrows0, semg0)

            drain(rows1, wrows1, semg1)

            @pl.when(kk > 0)
            def _():
                out_wait(out1, semo1)

            compute(rows1, wrows1, out1)
            pltpu.async_copy(
                out1, out_hbm.at[pl.ds(base2 + (ck0 + 1) * _E // 2, _E // 2)],
                semo1)

        out_wait(out0, semo0)
        out_wait(out1, semo1)

    return k(u_idx, w_idx, tab4)


def _tc_loss(packed):
    def body(x_ref, o_ref):
        x = x_ref[...]
        ev = lax.bitcast_convert_type(x << 16, jnp.float32)
        od = lax.bitcast_convert_type(
            x & jnp.int32(-65536), jnp.float32)
        s0 = jnp.sum(ev[:, :_DI] * ev[:, _DI:2 * _DI]
                     + od[:, :_DI] * od[:, _DI:2 * _DI],
                     axis=1, keepdims=True)
        s1 = jnp.sum(ev[:, 2 * _DI:3 * _DI] * ev[:, 3 * _DI:]
                     + od[:, 2 * _DI:3 * _DI] * od[:, 3 * _DI:],
                     axis=1, keepdims=True)
        row = lax.broadcasted_iota(jnp.int32, (_TOT // 2, 1), 0)
        sgn = jnp.where(row < _B // 2, -1.0, 1.0)
        ls = jax.nn.log_sigmoid(sgn * s0) + jax.nn.log_sigmoid(sgn * s1)
        o_ref[...] = jnp.sum(ls).reshape(1, 1)

    return pl.pallas_call(
        body,
        out_shape=jax.ShapeDtypeStruct((1, 1), jnp.float32),
    )(packed)


def kernel(pos_u, pos_w, neg_u, neg_w, n, u_table, w_table):
    u_idx, w_idx = _prep_indices(pos_u, pos_w, neg_u, neg_w)
    tab4 = _pack_tables(u_table, w_table)
    packed = _sc_gather_sum(u_idx, w_idx, tab4)
    loss = _tc_loss(packed)[0, 0]
    return -1.0 * loss / n


# speedup vs baseline: 10.3521x; 10.3521x over previous
import functools

import jax
import jax.numpy as jnp
from jax import lax
from jax.experimental import pallas as pl
from jax.experimental.pallas import tpu as pltpu
from jax.experimental.pallas import tpu_sc as plsc

_B = 16384
_CTX = 20
_D = 64
_DI = _D // 2
_TOT = 2 * _B
_NC, _NS = 2, 16
_NW = _NC * _NS
_PER_W = _TOT // _NW
_G = 128
_E = 32
_GPC = _E * _CTX // _G
_CHUNKS = _PER_W // _E
_LANES = 16
_ROWS = 199999


_HALF = (_ROWS + 1) // 2
_PB = 1000


def _pack_tables(u_table, w_table):
    def body(ulo, wlo, uhi, whi, o_ref):
        def bits(x):
            b = lax.bitcast_convert_type(x, jnp.uint32)
            return (b + jnp.uint32(0x7FFF) + ((b >> 16) & jnp.uint32(1))) >> 16

        def pk(x):
            return lax.bitcast_convert_type(
                bits(x[:, :_DI]) | (bits(x[:, _DI:]) << 16), jnp.int32)

        o_ref[...] = jnp.concatenate(
            [pk(ulo[...]), pk(wlo[...]), pk(uhi[...]), pk(whi[...])], axis=1)

    lo = pl.BlockSpec((_PB, _D), lambda i: (i, 0))
    hi = pl.BlockSpec((_PB, _D), lambda i: (i + _HALF // _PB, 0))
    comb = pl.pallas_call(
        body,
        grid=(_HALF // _PB,),
        in_specs=[lo, lo, hi, hi],
        out_specs=pl.BlockSpec((_PB, 4 * _DI), lambda i: (i, 0)),
        out_shape=jax.ShapeDtypeStruct((_HALF, 4 * _DI), jnp.int32),
    )(u_table, w_table, u_table, w_table)
    return comb.reshape(2 * (_ROWS + 1), _DI)


def _prep_indices(pos_u, pos_w, neg_u, neg_w):
    ui = jnp.concatenate(
        [pos_u.reshape(-1), neg_u.reshape(-1)]).astype(jnp.int32)
    wi = jnp.concatenate([pos_w, neg_w]).astype(jnp.int32)
    u_idx = (4 * (ui % _HALF) + 2 * (ui // _HALF)).reshape(
        _NW, _CHUNKS * _GPC, _G)
    w_idx = (4 * (wi % _HALF) + 2 * (wi // _HALF) + 1).reshape(
        _NW, _CHUNKS, _E)
    return u_idx, w_idx


def _sc_gather_sum(u_idx, w_idx, tab4):
    mesh = plsc.VectorSubcoreMesh(core_axis_name="c", subcore_axis_name="s")

    @functools.partial(
        pl.kernel,
        compiler_params=pltpu.CompilerParams(
            use_tc_tiling_on_sc=False, needs_layout_passes=False),
        out_type=jax.ShapeDtypeStruct((_TOT // 2, 4 * _DI), jnp.int32),
        mesh=mesh,
        scratch_types=[
            pltpu.VMEM((_CHUNKS * _GPC, _G), jnp.int32),
            pltpu.VMEM((_CHUNKS, _E), jnp.int32),
            pltpu.VMEM((_E * _CTX, _DI), jnp.int32),
            pltpu.VMEM((_E * _CTX, _DI), jnp.int32),
            pltpu.VMEM((_E, _DI), jnp.int32),
            pltpu.VMEM((_E, _DI), jnp.int32),
            pltpu.VMEM((_E // 2, 4 * _DI), jnp.int32),
            pltpu.VMEM((_E // 2, 4 * _DI), jnp.int32),
            pltpu.SemaphoreType.DMA,
            pltpu.SemaphoreType.DMA,
            pltpu.SemaphoreType.DMA,
            pltpu.SemaphoreType.DMA,
        ],
    )
    def k(uidx_hbm, widx_hbm, tab_hbm, out_hbm,
          uidx_v, widx_v, rows0, rows1, wrows0, wrows1, out0, out1,
          semg0, semg1, semo0, semo1):
        wid = lax.axis_index("s") * _NC + lax.axis_index("c")
        base2 = wid * _PER_W // 2
        pltpu.sync_copy(uidx_hbm.at[wid], uidx_v)
        pltpu.sync_copy(widx_hbm.at[wid], widx_v)

        def issue(ck, rows_v, wrows_v, semg):
            for j in range(_GPC):
                pltpu.async_copy(
                    tab_hbm.at[uidx_v.at[ck * _GPC + j]],
                    rows_v.at[pl.ds(j * _G, _G)],
                    semg,
                )
            pltpu.async_copy(tab_hbm.at[widx_v.at[ck]], wrows_v, semg)

        def drain(rows_v, wrows_v, semg):
            pltpu.make_async_copy(
                tab_hbm.at[pl.ds(0, _E * _CTX)], rows_v, semg).wait()
            pltpu.make_async_copy(tab_hbm.at[pl.ds(0, _E)], wrows_v, semg).wait()

        def compute(rows_v, wrows_v, out_v):
            @pl.loop(0, _E // 2)
            def _pair(p):
                for par in range(2):
                    r0 = (2 * p + par) * _CTX
                    ob = par * 2 * _DI
                    for h in range(_DI // _LANES):
                        sl = pl.ds(h * _LANES, _LANES)
                        acc = plsc.bitcast(rows_v[r0, sl], jnp.bfloat16)
                        for c in range(1, _CTX):
                            acc = acc + plsc.bitcast(
                                rows_v[r0 + c, sl], jnp.bfloat16)
                        out_v[p, pl.ds(ob + h * _LANES, _LANES)] = (
                            plsc.bitcast(acc, jnp.int32))
                        out_v[p, pl.ds(ob + _DI + h * _LANES, _LANES)] = (
                            wrows_v[2 * p + par, sl])

        def out_wait(out_v, semo):
            pltpu.make_async_copy(
                out_v, out_hbm.at[pl.ds(0, _E // 2)], semo).wait()

        _H = _CHUNKS // 2
        issue(0, rows0, wrows0, semg0)

        @pl.loop(0, _H)
        def _pipe(kk):
            ck0 = 2 * kk
            issue(ck0 + 1, rows1, wrows1, semg1)
            drain(rows0, wrows0, semg0)

            @pl.when(kk > 0)
            def _():
                out_wait(out0, semo0)

            compute(rows0, wrows0, out0)
            pltpu.async_copy(
                out0, out_hbm.at[pl.ds(base2 + ck0 * _E // 2, _E // 2)], semo0)

            @pl.when(kk < _H - 1)
            def _():
                issue(ck0 + 2, rows0, wrows0, semg0)

            drain(rows1, wrows1, semg1)

            @pl.when(kk > 0)
            def _():
                out_wait(out1, semo1)

            compute(rows1, wrows1, out1)
            pltpu.async_copy(
                out1, out_hbm.at[pl.ds(base2 + (ck0 + 1) * _E // 2, _E // 2)],
                semo1)

        out_wait(out0, semo0)
        out_wait(out1, semo1)

    return k(u_idx, w_idx, tab4)


def _tc_loss(packed):
    def body(x_ref, o_ref):
        x = x_ref[...]
        ev = lax.bitcast_convert_type(x << 16, jnp.float32)
        od = lax.bitcast_convert_type(
            x & jnp.int32(-65536), jnp.float32)
        s0 = jnp.sum(ev[:, :_DI] * ev[:, _DI:2 * _DI]
                     + od[:, :_DI] * od[:, _DI:2 * _DI],
                     axis=1, keepdims=True)
        s1 = jnp.sum(ev[:, 2 * _DI:3 * _DI] * ev[:, 3 * _DI:]
                     + od[:, 2 * _DI:3 * _DI] * od[:, 3 * _DI:],
                     axis=1, keepdims=True)
        row = lax.broadcasted_iota(jnp.int32, (_TOT // 2, 1), 0)
        sgn = jnp.where(row < _B // 2, -1.0, 1.0)
        ls = jax.nn.log_sigmoid(sgn * s0) + jax.nn.log_sigmoid(sgn * s1)
        o_ref[...] = jnp.sum(ls).reshape(1, 1)

    return pl.pallas_call(
        body,
        out_shape=jax.ShapeDtypeStruct((1, 1), jnp.float32),
    )(packed)


def kernel(pos_u, pos_w, neg_u, neg_w, n, u_table, w_table):
    u_idx, w_idx = _prep_indices(pos_u, pos_w, neg_u, neg_w)
    tab4 = _pack_tables(u_table, w_table)
    packed = _sc_gather_sum(u_idx, w_idx, tab4)
    loss = _tc_loss(packed)[0, 0]
    return -1.0 * loss / n


# speedup vs baseline: 13.5343x; 1.3074x over previous
import functools

import jax
import jax.numpy as jnp
from jax import lax
from jax.experimental import pallas as pl
from jax.experimental.pallas import tpu as pltpu
from jax.experimental.pallas import tpu_sc as plsc

_B = 16384
_CTX = 20
_D = 64
_TOT = 2 * _B
_NC, _NS = 2, 16
_NW = _NC * _NS
_PER_W = _TOT // _NW
_G = 128
_E = 32
_GPC = _E * _CTX // _G
_CHUNKS = _PER_W // _E
_DW = _D // 16
_LANES = 16
_ROWS = 199999


def _sc_gather_sum(u_idx, w_idx, tab2):
    mesh = plsc.VectorSubcoreMesh(core_axis_name="c", subcore_axis_name="s")

    @functools.partial(
        pl.kernel,
        compiler_params=pltpu.CompilerParams(use_tc_tiling_on_sc=False),
        out_type=jax.ShapeDtypeStruct((_TOT, 2 * _D), jnp.float32),
        mesh=mesh,
        scratch_types=[
            pltpu.VMEM((_CHUNKS * _GPC, _G), jnp.int32),
            pltpu.VMEM((_CHUNKS, _E), jnp.int32),
            pltpu.VMEM((_E * _CTX, _D), jnp.float32),
            pltpu.VMEM((_E * _CTX, _D), jnp.float32),
            pltpu.VMEM((_E, _D), jnp.float32),
            pltpu.VMEM((_E, _D), jnp.float32),
            pltpu.VMEM((_E, 2 * _D), jnp.float32),
            pltpu.VMEM((_E, 2 * _D), jnp.float32),
            pltpu.SemaphoreType.DMA,
            pltpu.SemaphoreType.DMA,
            pltpu.SemaphoreType.DMA,
            pltpu.SemaphoreType.DMA,
        ],
    )
    def k(uidx_hbm, widx_hbm, tab_hbm, out_hbm,
          uidx_v, widx_v, rows0, rows1, wrows0, wrows1, out0, out1,
          semg0, semg1, semo0, semo1):
        wid = lax.axis_index("s") * _NC + lax.axis_index("c")
        base = wid * _PER_W
        pltpu.sync_copy(uidx_hbm.at[wid], uidx_v)
        pltpu.sync_copy(widx_hbm.at[wid], widx_v)

        def issue(ck, rows_v, wrows_v, semg):
            for j in range(_GPC):
                pltpu.async_copy(
                    tab_hbm.at[uidx_v.at[ck * _GPC + j]],
                    rows_v.at[pl.ds(j * _G, _G)],
                    semg,
                )
            pltpu.async_copy(tab_hbm.at[widx_v.at[ck]], wrows_v, semg)

        def drain(rows_v, wrows_v, semg):
            pltpu.make_async_copy(
                tab_hbm.at[pl.ds(0, _E * _CTX)], rows_v, semg).wait()
            pltpu.make_async_copy(tab_hbm.at[pl.ds(0, _E)], wrows_v, semg).wait()

        def compute(rows_v, wrows_v, out_v):
            @pl.loop(0, _E)
            def _ex(e):
                r0 = e * _CTX
                for d in range(_DW):
                    sl = pl.ds(d * _LANES, _LANES)
                    acc = rows_v[r0, sl]
                    for c in range(1, _CTX):
                        acc = acc + rows_v[r0 + c, sl]
                    out_v[e, sl] = acc
                    out_v[e, pl.ds(_D + d * _LANES, _LANES)] = wrows_v[e, sl]

        def out_wait(out_v, semo):
            pltpu.make_async_copy(out_v, out_hbm.at[pl.ds(0, _E)], semo).wait()

        _H = _CHUNKS // 2
        issue(0, rows0, wrows0, semg0)

        @pl.loop(0, _H)
        def _pipe(kk):
            ck0 = 2 * kk
            issue(ck0 + 1, rows1, wrows1, semg1)
            drain(rows0, wrows0, semg0)

            @pl.when(kk > 0)
            def _():
                out_wait(out0, semo0)

            compute(rows0, wrows0, out0)
            pltpu.async_copy(out0, out_hbm.at[pl.ds(base + ck0 * _E, _E)], semo0)

            @pl.when(kk < _H - 1)
            def _():
                issue(ck0 + 2, rows0, wrows0, semg0)

            drain(rows1, wrows1, semg1)

            @pl.when(kk > 0)
            def _():
                out_wait(out1, semo1)

            compute(rows1, wrows1, out1)
            pltpu.async_copy(
                out1, out_hbm.at[pl.ds(base + (ck0 + 1) * _E, _E)], semo1)

        out_wait(out0, semo0)
        out_wait(out1, semo1)

    return k(u_idx, w_idx, tab2)


def _tc_loss(uw_emb):
    def body(x_ref, o_ref):
        u = x_ref[:, : _D]
        w = x_ref[:, _D:]
        s = jnp.sum(u * w, axis=1, keepdims=True)
        row = lax.broadcasted_iota(jnp.int32, (_TOT, 1), 0)
        z = jnp.where(row < _B, -s, s)
        o_ref[...] = jnp.sum(jax.nn.log_sigmoid(z)).reshape(1, 1)

    return pl.pallas_call(
        body,
        out_shape=jax.ShapeDtypeStruct((1, 1), jnp.float32),
    )(uw_emb)


def kernel(pos_u, pos_w, neg_u, neg_w, n, u_table, w_table):
    u_idx = (2 * jnp.concatenate(
        [pos_u.reshape(-1), neg_u.reshape(-1)]
    ).astype(jnp.int32)).reshape(_NW, _CHUNKS * _GPC, _G)
    w_idx = (2 * jnp.concatenate([pos_w, neg_w]).astype(jnp.int32)
             + 1).reshape(_NW, _CHUNKS, _E)
    comb = jnp.concatenate(
        [jnp.pad(u_table, ((0, 1), (0, 0))), jnp.pad(w_table, ((0, 1), (0, 0)))],
        axis=1,
    )
    tab2 = comb.reshape(2 * (_ROWS + 1), _D)
    uw_emb = _sc_gather_sum(u_idx, w_idx, tab2)
    loss = _tc_loss(uw_emb)[0, 0]
    return -1.0 * loss / n
